# baseline (speedup 1.0000x reference)
import jax
import jax.numpy as jnp
from jax import lax
from jax.experimental import pallas as pl
from jax.experimental.pallas import tpu as pltpu
from jax.experimental.pallas import tpu_sc as plsc

_L = 16
_NW = 32
_B = 16384
_LEN = 100000
_D = 64
_NB = 4
_W = 512
_NC = 196
_SLOTS = 7
_NOFF = 256
_DUMP = _NB * _B


def _iota16():
    return lax.iota(jnp.int32, _L)


def _body(inp_hbm, gt_hbm, sidx_hbm, spos_hbm, offs_hbm,
          outx_hbm, outg_hbm,
          sidx_v, spos_v, offs_v, slab, rows_a, rows_b, pos_a, pos_b,
          sem_in, sem_out):
    wid = lax.axis_index("s") * 2 + lax.axis_index("c")

    pltpu.sync_copy(sidx_hbm, sidx_v)
    pltpu.sync_copy(spos_hbm, spos_v)
    pltpu.sync_copy(offs_hbm, offs_v)

    rbuf = rows_a
    pbuf = pos_a
    del rows_b, pos_b

    def slot_body(s, carry0):
        c = wid + s * jnp.int32(_NW)

        @pl.when(c < jnp.int32(_NC))
        def _():
            lo = jnp.where(jnp.equal(c, _NC - 1), jnp.int32(99584), c * _W)
            ovec = offs_v[pl.ds(c, _L)]
            start = ovec[0]
            end = ovec[1]
            ngroups = lax.div(end - start + jnp.int32(_L - 1), jnp.int32(_L))

            def bt_body(tab, out):
                def b_body(b, carry1):
                    pltpu.async_copy(tab.at[b, :, pl.ds(lo, _W)], slab,
                                     sem_in).wait()

                    def ext_body(g, carry2):
                        at = start + g * _L
                        v = sidx_v[pl.ds(at, _L)]
                        p = spos_v[pl.ds(at, _L)]
                        valid = at + _iota16() < end
                        lvec = jnp.minimum(
                            jnp.maximum(v - lo, jnp.int32(0)),
                            jnp.int32(_W - 1))
                        pbuf[...] = jnp.where(valid, p + b * jnp.int32(_B),
                                              jnp.int32(_DUMP) + wid)
                        for d in range(_D):
                            w = plsc.load_gather(
                                slab, [jnp.full((_L,), d, jnp.int32), lvec])
                            plsc.store_scatter(
                                rbuf, [_iota16(),
                                       jnp.full((_L,), d, jnp.int32)], w)
                        pltpu.async_copy(rbuf, out.at[pbuf], sem_out).wait()
                        return carry2

                    lax.fori_loop(0, ngroups, ext_body, jnp.int32(0))
                    return carry1

                lax.fori_loop(0, _NB, b_body, jnp.int32(0))

            bt_body(inp_hbm, outx_hbm)
            bt_body(gt_hbm, outg_hbm)

        return carry0

    lax.fori_loop(0, _SLOTS, slot_body, jnp.int32(0))


@jax.jit
def kernel(inp, gt, xs):
    inp_t = jnp.swapaxes(inp, 1, 2)
    gt_t = jnp.swapaxes(gt, 1, 2)

    idx = jnp.clip((xs * jnp.float32(_LEN)).astype(jnp.int32), 0, _LEN - 1)
    order = jnp.argsort(idx)
    sidx = idx[order]
    spos = order.astype(jnp.int32)
    bounds = (jnp.arange(_NOFF, dtype=jnp.int32) * _W).clip(0, _LEN)
    offs = jnp.searchsorted(sidx, bounds).astype(jnp.int32)

    mesh = plsc.VectorSubcoreMesh(core_axis_name="c", subcore_axis_name="s")
    nrows = _NB * _B + _NW
    out_type = (jax.ShapeDtypeStruct((nrows, 2 * _D), jnp.float32),
                jax.ShapeDtypeStruct((nrows, 2 * _D), jnp.float32))
    run = pl.kernel(
        _body,
        out_type=out_type,
        mesh=mesh,
        scratch_types=[
            pltpu.VMEM((_B,), jnp.int32),
            pltpu.VMEM((_B,), jnp.int32),
            pltpu.VMEM((_NOFF,), jnp.int32),
            pltpu.VMEM((_D, _W), jnp.float32),
            pltpu.VMEM((_L, 2 * _D), jnp.float32),
            pltpu.VMEM((_L, 2 * _D), jnp.float32),
            pltpu.VMEM((_L,), jnp.int32),
            pltpu.VMEM((_L,), jnp.int32),
            pltpu.SemaphoreType.DMA,
            pltpu.SemaphoreType.DMA,
        ],
        compiler_params=pltpu.CompilerParams(use_tc_tiling_on_sc=True, needs_layout_passes=False),
    )
    ox, og = run(inp_t, gt_t, sidx, spos, offs)
    return ox[:_NB * _B, :_D], og[:_NB * _B, :_D]


# speedup vs baseline: 1.0834x; 1.0834x over previous
import jax
import jax.numpy as jnp
from jax import lax
from jax.experimental import pallas as pl
from jax.experimental.pallas import tpu as pltpu
from jax.experimental.pallas import tpu_sc as plsc

_L = 16
_NW = 32
_B = 16384
_LEN = 100000
_D = 64
_NB = 4
_W = 512
_NC = 196
_SLOTS = 7
_NOFF = 256
_DUMP = _NB * _B
_RING = 8


def _iota16():
    return lax.iota(jnp.int32, _L)


def _body(inp_hbm, gt_hbm, sidx_hbm, spos_hbm, offs_hbm,
          outx_hbm, outg_hbm,
          sidx_v, spos_v, offs_v, slab, ringr, ringp,
          sem_in, sem_out):
    wid = lax.axis_index("s") * 2 + lax.axis_index("c")

    pltpu.sync_copy(sidx_hbm, sidx_v)
    pltpu.sync_copy(spos_hbm, spos_v)
    pltpu.sync_copy(offs_hbm, offs_v)

    def slot_body(s, carry0):
        c = wid + s * jnp.int32(_NW)

        @pl.when(c < jnp.int32(_NC))
        def _():
            lo = jnp.where(jnp.equal(c, _NC - 1), jnp.int32(99584), c * _W)
            ovec = offs_v[pl.ds(c, _L)]
            start = ovec[0]
            end = ovec[1]
            ngroups = lax.div(end - start + jnp.int32(_L - 1), jnp.int32(_L))

            def bt_body(tab, out):
                def b_body(b, carry1):
                    pltpu.async_copy(tab.at[b, :, pl.ds(lo, _W)], slab,
                                     sem_in).wait()

                    def ext_body(g, carry2):
                        slot = jnp.bitwise_and(g, jnp.int32(_RING - 1))
                        rbuf = ringr.at[slot]
                        pbuf = ringp.at[slot]

                        @pl.when(g >= jnp.int32(_RING))
                        def _():
                            pltpu.make_async_copy(
                                out.at[pl.ds(0, _L)], ringr.at[0],
                                sem_out).wait()

                        at = start + g * _L
                        v = sidx_v[pl.ds(at, _L)]
                        p = spos_v[pl.ds(at, _L)]
                        valid = at + _iota16() < end
                        lvec = jnp.minimum(
                            jnp.maximum(v - lo, jnp.int32(0)),
                            jnp.int32(_W - 1))
                        pbuf[...] = jnp.where(valid, p + b * jnp.int32(_B),
                                              jnp.int32(_DUMP) + wid)
                        for d in range(_D):
                            w = plsc.load_gather(
                                slab, [jnp.full((_L,), d, jnp.int32), lvec])
                            plsc.store_scatter(
                                rbuf, [_iota16(),
                                       jnp.full((_L,), d, jnp.int32)], w)
                        pltpu.async_copy(rbuf, out.at[pbuf], sem_out)
                        return carry2

                    lax.fori_loop(0, ngroups, ext_body, jnp.int32(0))

                    def drain_body(j, carry2):
                        pltpu.make_async_copy(
                            out.at[pl.ds(0, _L)], ringr.at[0],
                            sem_out).wait()
                        return carry2

                    lax.fori_loop(0, jnp.minimum(ngroups, jnp.int32(_RING)),
                                  drain_body, jnp.int32(0))
                    return carry1

                lax.fori_loop(0, _NB, b_body, jnp.int32(0))

            bt_body(inp_hbm, outx_hbm)
            bt_body(gt_hbm, outg_hbm)

        return carry0

    lax.fori_loop(0, _SLOTS, slot_body, jnp.int32(0))


@jax.jit
def kernel(inp, gt, xs):
    inp_t = jnp.swapaxes(inp, 1, 2)
    gt_t = jnp.swapaxes(gt, 1, 2)

    idx = jnp.clip((xs * jnp.float32(_LEN)).astype(jnp.int32), 0, _LEN - 1)
    order = jnp.argsort(idx)
    sidx = idx[order]
    spos = order.astype(jnp.int32)
    bounds = (jnp.arange(_NOFF, dtype=jnp.int32) * _W).clip(0, _LEN)
    offs = jnp.searchsorted(sidx, bounds).astype(jnp.int32)

    mesh = plsc.VectorSubcoreMesh(core_axis_name="c", subcore_axis_name="s")
    nrows = _NB * _B + _NW
    out_type = (jax.ShapeDtypeStruct((nrows, 2 * _D), jnp.float32),
                jax.ShapeDtypeStruct((nrows, 2 * _D), jnp.float32))
    run = pl.kernel(
        _body,
        out_type=out_type,
        mesh=mesh,
        scratch_types=[
            pltpu.VMEM((_B,), jnp.int32),
            pltpu.VMEM((_B,), jnp.int32),
            pltpu.VMEM((_NOFF,), jnp.int32),
            pltpu.VMEM((_D, _W), jnp.float32),
            pltpu.VMEM((_RING, _L, 2 * _D), jnp.float32),
            pltpu.VMEM((_RING, _L), jnp.int32),
            pltpu.SemaphoreType.DMA,
            pltpu.SemaphoreType.DMA,
        ],
        compiler_params=pltpu.CompilerParams(use_tc_tiling_on_sc=True, needs_layout_passes=False),
    )
    ox, og = run(inp_t, gt_t, sidx, spos, offs)
    return ox[:_NB * _B, :_D], og[:_NB * _B, :_D]
